# baseline (speedup 1.0000x reference)
import functools

import jax
import jax.numpy as jnp
from jax import lax
from jax.experimental import pallas as pl
from jax.experimental.pallas import tpu as pltpu
from jax.experimental.pallas import tpu_sc as plsc

NC = 2
NS = 16
NW = NC * NS
CHUNK = 128
GM = 3


def _spmm_partials(x, src2d, dst2d, zeros):
    d = x.shape[1]
    npad = zeros.shape[0]
    r = src2d.shape[0]
    rows_per_sub = npad // NS
    iters = (r + NW - 1) // NW
    mesh = plsc.VectorSubcoreMesh(core_axis_name="c", subcore_axis_name="s")

    @functools.partial(
        pl.kernel,
        out_type=jax.ShapeDtypeStruct((NC, npad, d), jnp.float32),
        mesh=mesh,
        scratch_types=[
            pltpu.VMEM((GM * CHUNK,), jnp.int32),
            [pltpu.VMEM((CHUNK,), jnp.int32) for _ in range(GM)],
            pltpu.VMEM((GM * CHUNK, d), jnp.float32),
            pltpu.VMEM_SHARED((npad, d), jnp.float32),
            pltpu.SemaphoreType.DMA,
            pltpu.SemaphoreType.DMA,
        ],
    )
    def spmm(x_hbm, src_hbm, dst_hbm, zero_hbm, out_hbm,
             sidx, didx, rows, yacc, sem, isem):
        c = lax.axis_index("c")
        s = lax.axis_index("s")
        wid = s * NC + c
        pltpu.sync_copy(zero_hbm.at[pl.ds(s * rows_per_sub, rows_per_sub)],
                        yacc.at[pl.ds(s * rows_per_sub, rows_per_sub)])
        plsc.subcore_barrier()

        pltpu.async_copy(src_hbm.at[wid], sidx, isem)

        def body(it, carry):
            row = wid + it * NW

            @pl.when(row < r)
            def _():
                pltpu.make_async_copy(src_hbm.at[row], sidx, isem).wait()
                gat = pltpu.async_copy(x_hbm.at[sidx], rows, sem)
                for k in range(GM):
                    pltpu.sync_copy(dst_hbm.at[GM * row + k], didx[k])
                gat.wait()
                nrow = row + NW

                @pl.when(nrow < r)
                def _p():
                    pltpu.async_copy(src_hbm.at[nrow], sidx, isem)

                for k in range(GM):
                    pltpu.sync_copy(rows.at[pl.ds(k * CHUNK, CHUNK)],
                                    yacc.at[didx[k]], add=True)

            return carry

        lax.fori_loop(0, iters, body, 0)
        plsc.subcore_barrier()
        pltpu.sync_copy(yacc.at[pl.ds(s * rows_per_sub, rows_per_sub)],
                        out_hbm.at[c, pl.ds(s * rows_per_sub, rows_per_sub)])

    return spmm(x, src2d, dst2d, zeros)


def _mlp_body(y_ref, x_ref, w1_ref, b1_ref, w2_ref, b2_ref, scale_ref,
              g_ref, bt_ref, o_ref):
    n = x_ref.shape[0]
    h = y_ref[0, :n] + y_ref[1, :n] + scale_ref[...] * x_ref[...]
    h = jnp.dot(h, w1_ref[...], preferred_element_type=jnp.float32)
    h = jnp.maximum(h + b1_ref[...], 0.0)
    h = jnp.dot(h, w2_ref[...], preferred_element_type=jnp.float32)
    h = h + b2_ref[...]
    mean = jnp.mean(h, axis=0, keepdims=True)
    var = jnp.mean(jnp.square(h - mean), axis=0, keepdims=True)
    h = (h - mean) * lax.rsqrt(var + 1e-5) * g_ref[...] + bt_ref[...]
    o_ref[...] = jnp.maximum(h, 0.0)


def kernel(x, edge_index, W1, b1, W2, b2, eps, gamma, beta):
    n, d = x.shape
    e = edge_index.shape[1]
    assert e % CHUNK == 0
    npad = ((n + 8 * NS - 1) // (8 * NS)) * (8 * NS)
    gtile = GM * CHUNK
    epad = ((e + gtile - 1) // gtile) * gtile
    dst1 = edge_index[0]
    src1 = edge_index[1]
    if epad != e:
        dst1 = jnp.concatenate(
            [dst1, jnp.full((epad - e,), npad - 1, jnp.int32)])
        src1 = jnp.concatenate([src1, jnp.zeros((epad - e,), jnp.int32)])
    dst2d = dst1.reshape(epad // CHUNK, CHUNK)
    src2d = src1.reshape(epad // gtile, gtile)
    zeros = jnp.zeros((npad, d), jnp.float32)

    partials = _spmm_partials(x, src2d, dst2d, zeros)

    scale = (1.0 + eps).reshape(1, 1)
    out = pl.pallas_call(
        _mlp_body,
        out_shape=jax.ShapeDtypeStruct((n, d), jnp.float32),
    )(partials, x, W1.T, b1.reshape(1, d), W2.T, b2.reshape(1, d),
      scale, gamma.reshape(1, d), beta.reshape(1, d))
    return out


# speedup vs baseline: 1.0735x; 1.0735x over previous
import functools

import jax
import jax.numpy as jnp
from jax import lax
from jax.experimental import pallas as pl
from jax.experimental.pallas import tpu as pltpu
from jax.experimental.pallas import tpu_sc as plsc

NC = 2
NS = 16
NW = NC * NS
CHUNK = 128
GM = 2


def _spmm_partials(x, src2d, dst2d, zeros):
    d = x.shape[1]
    npad = zeros.shape[0]
    r = src2d.shape[0]
    rows_per_sub = npad // NS
    iters = (r + NW - 1) // NW
    mesh = plsc.VectorSubcoreMesh(core_axis_name="c", subcore_axis_name="s")

    @functools.partial(
        pl.kernel,
        out_type=jax.ShapeDtypeStruct((NC, npad, d), jnp.float32),
        mesh=mesh,
        scratch_types=[
            pltpu.VMEM((GM * CHUNK,), jnp.int32),
            [pltpu.VMEM((CHUNK,), jnp.int32) for _ in range(GM)],
            pltpu.VMEM((GM * CHUNK, d), jnp.float32),
            pltpu.VMEM_SHARED((npad, d), jnp.float32),
            pltpu.SemaphoreType.DMA,
            pltpu.SemaphoreType.DMA,
        ],
    )
    def spmm(x_hbm, src_hbm, dst_hbm, zero_hbm, out_hbm,
             sidx, didx, rows, yacc, sem, isem):
        c = lax.axis_index("c")
        s = lax.axis_index("s")
        wid = s * NC + c
        pltpu.sync_copy(zero_hbm.at[pl.ds(s * rows_per_sub, rows_per_sub)],
                        yacc.at[pl.ds(s * rows_per_sub, rows_per_sub)])
        plsc.subcore_barrier()

        pltpu.async_copy(src_hbm.at[wid], sidx, isem)

        def body(it, carry):
            row = wid + it * NW

            @pl.when(row < r)
            def _():
                pltpu.make_async_copy(src_hbm.at[row], sidx, isem).wait()
                gat = pltpu.async_copy(x_hbm.at[sidx], rows, sem)
                for k in range(GM):
                    pltpu.sync_copy(dst_hbm.at[GM * row + k], didx[k])
                gat.wait()
                nrow = row + NW

                @pl.when(nrow < r)
                def _p():
                    pltpu.async_copy(src_hbm.at[nrow], sidx, isem)

                for k in range(GM):
                    pltpu.sync_copy(rows.at[pl.ds(k * CHUNK, CHUNK)],
                                    yacc.at[didx[k]], add=True)

            return carry

        lax.fori_loop(0, iters, body, 0)
        plsc.subcore_barrier()
        pltpu.sync_copy(yacc.at[pl.ds(s * rows_per_sub, rows_per_sub)],
                        out_hbm.at[c, pl.ds(s * rows_per_sub, rows_per_sub)])

    return spmm(x, src2d, dst2d, zeros)


def _mlp_body(y_ref, x_ref, w1_ref, b1_ref, w2_ref, b2_ref, scale_ref,
              g_ref, bt_ref, o_ref):
    n = x_ref.shape[0]
    h = y_ref[0, :n] + y_ref[1, :n] + scale_ref[...] * x_ref[...]
    h = jnp.dot(h, w1_ref[...], preferred_element_type=jnp.float32)
    h = jnp.maximum(h + b1_ref[...], 0.0)
    h = jnp.dot(h, w2_ref[...], preferred_element_type=jnp.float32)
    h = h + b2_ref[...]
    mean = jnp.mean(h, axis=0, keepdims=True)
    var = jnp.mean(jnp.square(h - mean), axis=0, keepdims=True)
    h = (h - mean) * lax.rsqrt(var + 1e-5) * g_ref[...] + bt_ref[...]
    o_ref[...] = jnp.maximum(h, 0.0)


def kernel(x, edge_index, W1, b1, W2, b2, eps, gamma, beta):
    n, d = x.shape
    e = edge_index.shape[1]
    assert e % CHUNK == 0
    npad = ((n + 8 * NS - 1) // (8 * NS)) * (8 * NS)
    gtile = GM * CHUNK
    epad = ((e + gtile - 1) // gtile) * gtile
    dst1 = edge_index[0]
    src1 = edge_index[1]
    if epad != e:
        dst1 = jnp.concatenate(
            [dst1, jnp.full((epad - e,), npad - 1, jnp.int32)])
        src1 = jnp.concatenate([src1, jnp.zeros((epad - e,), jnp.int32)])
    dst2d = dst1.reshape(epad // CHUNK, CHUNK)
    src2d = src1.reshape(epad // gtile, gtile)
    zeros = jnp.zeros((npad, d), jnp.float32)

    partials = _spmm_partials(x, src2d, dst2d, zeros)

    scale = (1.0 + eps).reshape(1, 1)
    out = pl.pallas_call(
        _mlp_body,
        out_shape=jax.ShapeDtypeStruct((n, d), jnp.float32),
    )(partials, x, W1.T, b1.reshape(1, d), W2.T, b2.reshape(1, d),
      scale, gamma.reshape(1, d), beta.reshape(1, d))
    return out
